# Initial kernel scaffold; baseline (speedup 1.0000x reference)
import jax
import jax.numpy as jnp
from jax.experimental import pallas as pl


def kernel(nodes_memory, crowds_memory, interests_memory, categories_memory, brands_memory, values1, values2, values3, values4, values5, users_idxs):
    raise NotImplementedError("write your pallas kernel here")


# speedup vs baseline: 3.4827x; 3.4827x over previous
import functools

import jax
import jax.numpy as jnp
from jax import lax
from jax.experimental import pallas as pl
from jax.experimental.pallas import tpu as pltpu
from jax.experimental.pallas import tpu_sc as plsc

N = 100000
D = 64
B = 16384
L = 16
NC = 2
NS = 16
NW = NC * NS
BPW = B // NW
NVEC = B // L
CHUNK = 128
NCHUNK = BPW // CHUNK

_mesh = plsc.VectorSubcoreMesh(core_axis_name="c", subcore_axis_name="s")


@functools.partial(
    pl.kernel,
    out_type=jax.ShapeDtypeStruct((B,), jnp.int32),
    mesh=_mesh,
    compiler_params=pltpu.CompilerParams(
        needs_layout_passes=False, use_tc_tiling_on_sc=False),
    scratch_types=[
        pltpu.VMEM((N,), jnp.int32),
        pltpu.VMEM((B,), jnp.int32),
        pltpu.VMEM((L,), jnp.int32),
    ],
)
def _last_writer(idx_hbm, m_hbm, pos, xm, scr):
    core = lax.axis_index("c")
    sub = lax.axis_index("s")

    @pl.when(jnp.logical_and(core == 0, sub == 0))
    def _():
        pltpu.sync_copy(idx_hbm, xm)
        lane = lax.iota(jnp.int32, 16)
        nxt_lane = jnp.minimum(lane + 1, 15)
        is_last_lane = lane == 15

        def scatter_body(c, carry):
            x = xm[pl.ds(c * L, L)]
            comp = (x << 14) | (lane + c * L)
            s, _ = plsc.sort_key_val(comp, comp)
            scr[...] = s
            s_nxt = plsc.load_gather(scr, [nxt_lane])
            keep = ((s >> 14) != (s_nxt >> 14)) | is_last_lane
            plsc.store_scatter(pos, [s >> 14], s & 16383, mask=keep)
            return carry

        lax.fori_loop(0, NVEC, scatter_body, 0)

        def gather_body(c, carry):
            x = xm[pl.ds(c * L, L)]
            xm[pl.ds(c * L, L)] = plsc.load_gather(pos, [x])
            return carry

        lax.fori_loop(0, NVEC, gather_body, 0)
        pltpu.sync_copy(xm, m_hbm)


@functools.partial(
    pl.kernel,
    out_type=tuple(jax.ShapeDtypeStruct((B, D), jnp.float32) for _ in range(5)),
    mesh=_mesh,
    compiler_params=pltpu.CompilerParams(
        needs_layout_passes=False, use_tc_tiling_on_sc=False),
    scratch_types=[
        pltpu.VMEM((NCHUNK, CHUNK), jnp.int32),
        pltpu.VMEM((CHUNK, D), jnp.float32),
        pltpu.VMEM((CHUNK, D), jnp.float32),
        pltpu.SemaphoreType.DMA,
        pltpu.SemaphoreType.DMA,
    ],
)
def _gather5(m_hbm, v1, v2, v3, v4, v5, o1, o2, o3, o4, o5,
             m_v, buf_a, buf_b, sem_a, sem_b):
    core = lax.axis_index("c")
    sub = lax.axis_index("s")
    wid = sub * NC + core
    base = wid * BPW

    for j in range(NCHUNK):
        pltpu.sync_copy(m_hbm.at[pl.ds(base + j * CHUNK, CHUNK)], m_v.at[j])

    vs = (v1, v2, v3, v4, v5)
    os_ = (o1, o2, o3, o4, o5)
    bufs = (buf_a, buf_b)
    sems = (sem_a, sem_b)
    steps = [(k, j) for k in range(5) for j in range(NCHUNK)]

    def fire(t):
        k, j = steps[t]
        return pltpu.async_copy(vs[k].at[m_v.at[j]], bufs[t % 2], sems[t % 2])

    cp = fire(0)
    for t in range(len(steps)):
        nxt = fire(t + 1) if t + 1 < len(steps) else None
        cp.wait()
        k, j = steps[t]
        pltpu.sync_copy(bufs[t % 2], os_[k].at[pl.ds(base + j * CHUNK, CHUNK)])
        cp = nxt


def kernel(nodes_memory, crowds_memory, interests_memory, categories_memory,
           brands_memory, values1, values2, values3, values4, values5,
           users_idxs):
    m = _last_writer(users_idxs)
    return _gather5(m, values1, values2, values3, values4, values5)
